# baseline (speedup 1.0000x reference)
import jax
import jax.numpy as jnp
from jax import lax
from jax.experimental import pallas as pl
from jax.experimental.pallas import tpu as pltpu
from jax.experimental.pallas import tpu_sc as plsc

_B, _T, _D = 16, 512, 384
_L = 2048
_LANES = 16
_NC, _NS = 2, 16
_HALF = _L // 2
_CHUNK = 64
_NCHUNK = _HALF // _CHUNK
_NB = 4
_BSTEPS = 10


def _lr_body(x_hbm, dur_hbm, out_hbm, mel_hbm,
             dur_v, cs_v, idx_v, rows0_v, rows1_v, rows2_v, rows3_v,
             mel_v,
             gsem0, gsem1, gsem2, gsem3, ssem0, ssem1, ssem2, ssem3):
    cid = lax.axis_index("c")
    sid = lax.axis_index("s")
    wid = sid * _NC + cid
    b = wid % _B
    half = wid // _B
    lo = half * _HALF
    lanes = lax.iota(jnp.int32, _LANES)

    pltpu.sync_copy(dur_hbm.at[b], dur_v)

    def cs_body(i, carry):
        v = dur_v[pl.ds(i * _LANES, _LANES)]
        cs_v[pl.ds(i * _LANES, _LANES)] = plsc.cumsum(v) + carry
        return carry + jnp.sum(v)

    total = lax.fori_loop(0, _T // _LANES, cs_body, jnp.int32(0))
    cs_v[pl.ds(_T, _LANES)] = jnp.full((_LANES,), 2**30, jnp.int32)

    def bisect_chunk(g):
        def vec_idx(k, _):
            j = (lo + g * _CHUNK + k * _LANES) + lanes
            lov = jnp.zeros((_LANES,), jnp.int32)
            hiv = jnp.full((_LANES,), _T, jnp.int32)
            for _s in range(_BSTEPS):
                mid = (lov + hiv) >> 1
                vv = plsc.load_gather(cs_v, [mid])
                le = vv <= j
                lov = jnp.where(le, mid + 1, lov)
                hiv = jnp.where(le, hiv, mid)
            idx_v[2 * g + k // 2, pl.ds((k % 2) * _LANES, _LANES)] = (
                b * _T + jnp.minimum(lov, _T - 1))
            return 0

        lax.fori_loop(0, _CHUNK // _LANES, vec_idx, 0)

    bufs = (rows0_v, rows1_v, rows2_v, rows3_v)
    gsems = (gsem0, gsem1, gsem2, gsem3)
    ssems = (ssem0, ssem1, ssem2, ssem3)

    _HC = _CHUNK // 2

    def start_gather(c, i):
        d1 = pltpu.async_copy(
            x_hbm.at[idx_v.at[2 * c]], bufs[i].at[pl.ds(0, _HC)], gsems[i])
        d2 = pltpu.async_copy(
            x_hbm.at[idx_v.at[2 * c + 1]], bufs[i].at[pl.ds(_HC, _HC)],
            gsems[i])
        return (d1, d2)

    def out_slice(c):
        return out_hbm.at[b, pl.ds(lo + c * _CHUNK, _CHUNK)]

    gs = [None] * _NB
    stores = [None] * _NB

    for g in range(2):
        bisect_chunk(g)
        gs[g % _NB] = start_gather(g, g % _NB)

    cap = jnp.minimum(total, _L)
    nvalid = jnp.clip(cap - lo, 0, _HALF)

    def _mel():
        mel_v[...] = jnp.zeros((_LANES,), jnp.int32) + total
        pltpu.sync_copy(mel_v, mel_hbm.at[b])
    pl.when(half == 0)(_mel)

    for c in range(_NCHUNK):
        i = c % _NB
        g = c + 2
        if g < _NCHUNK:
            jbuf = g % _NB
            if stores[jbuf] is not None:
                stores[jbuf].wait()
            bisect_chunk(g)
            gs[jbuf] = start_gather(g, jbuf)
        gs[i][0].wait()
        gs[i][1].wait()
        nv_c = jnp.clip(nvalid - c * _CHUNK, 0, _CHUNK)

        def z_body(r, _, _i=i):
            zer = jnp.zeros((_LANES,), jnp.float32)
            for q in range(_D // _LANES):
                bufs[_i][r, pl.ds(q * _LANES, _LANES)] = zer
            return 0

        lax.fori_loop(nv_c, _CHUNK, z_body, 0)
        stores[i] = pltpu.async_copy(bufs[i], out_slice(c), ssems[i])
    for i in range(_NB):
        stores[i].wait()


def kernel(x, duration, max_len):
    del max_len
    x2 = x.reshape(_B * _T, _D)
    mesh = plsc.VectorSubcoreMesh(
        core_axis_name="c", subcore_axis_name="s",
        num_cores=_NC, num_subcores=_NS)
    f = pl.kernel(
        _lr_body,
        out_type=(
            jax.ShapeDtypeStruct((_B, _L, _D), jnp.float32),
            jax.ShapeDtypeStruct((_B, _LANES), jnp.int32),
        ),
        mesh=mesh,
        compiler_params=pltpu.CompilerParams(needs_layout_passes=False),
        scratch_types=[
            pltpu.VMEM((_T,), jnp.int32),
            pltpu.VMEM((_T + _LANES,), jnp.int32),
            pltpu.VMEM((_NCHUNK * 2, _CHUNK // 2), jnp.int32),
            pltpu.VMEM((_CHUNK, _D), jnp.float32),
            pltpu.VMEM((_CHUNK, _D), jnp.float32),
            pltpu.VMEM((_CHUNK, _D), jnp.float32),
            pltpu.VMEM((_CHUNK, _D), jnp.float32),
            pltpu.VMEM((_LANES,), jnp.int32),
            pltpu.SemaphoreType.DMA,
            pltpu.SemaphoreType.DMA,
            pltpu.SemaphoreType.DMA,
            pltpu.SemaphoreType.DMA,
            pltpu.SemaphoreType.DMA,
            pltpu.SemaphoreType.DMA,
            pltpu.SemaphoreType.DMA,
            pltpu.SemaphoreType.DMA,
        ],
    )
    out, mel2d = f(x2, duration)
    return out, mel2d[:, 0]


# speedup vs baseline: 1.0067x; 1.0067x over previous
import jax
import jax.numpy as jnp
from jax import lax
from jax.experimental import pallas as pl
from jax.experimental.pallas import tpu as pltpu
from jax.experimental.pallas import tpu_sc as plsc

_B, _T, _D = 16, 512, 384
_L = 2048
_LANES = 16
_NC, _NS = 2, 16
_HALF = _L // 2
_CHUNK = 64
_NCHUNK = _HALF // _CHUNK
_NB = 4
_BSTEPS = 10


def _lr_body(x_hbm, dur_hbm, ml_hbm, out_hbm, mel_hbm,
             dur_v, cs_v, idx_v, rows0_v, rows1_v, rows2_v, rows3_v,
             mlv_v, mel_v,
             gsem0, gsem1, gsem2, gsem3, ssem0, ssem1, ssem2, ssem3):
    cid = lax.axis_index("c")
    sid = lax.axis_index("s")
    wid = sid * _NC + cid
    b = wid % _B
    half = wid // _B
    lo = half * _HALF
    lanes = lax.iota(jnp.int32, _LANES)

    pltpu.sync_copy(dur_hbm.at[b], dur_v)

    def cs_body(i, carry):
        v = dur_v[pl.ds(i * _LANES, _LANES)]
        cs_v[pl.ds(i * _LANES, _LANES)] = plsc.cumsum(v) + carry
        return carry + jnp.sum(v)

    total = lax.fori_loop(0, _T // _LANES, cs_body, jnp.int32(0))
    cs_v[pl.ds(_T, _LANES)] = jnp.full((_LANES,), 2**30, jnp.int32)

    def bisect_chunk(g):
        def vec_idx(k, _):
            j = (lo + g * _CHUNK + k * _LANES) + lanes
            lov = jnp.zeros((_LANES,), jnp.int32)
            hiv = jnp.full((_LANES,), _T, jnp.int32)
            for _s in range(_BSTEPS):
                mid = (lov + hiv) >> 1
                vv = plsc.load_gather(cs_v, [mid])
                le = vv <= j
                lov = jnp.where(le, mid + 1, lov)
                hiv = jnp.where(le, hiv, mid)
            idx_v[2 * g + k // 2, pl.ds((k % 2) * _LANES, _LANES)] = (
                b * _T + jnp.minimum(lov, _T - 1))
            return 0

        lax.fori_loop(0, _CHUNK // _LANES, vec_idx, 0)

    bufs = (rows0_v, rows1_v, rows2_v, rows3_v)
    gsems = (gsem0, gsem1, gsem2, gsem3)
    ssems = (ssem0, ssem1, ssem2, ssem3)

    _HC = _CHUNK // 2

    def start_gather(c, i):
        d1 = pltpu.async_copy(
            x_hbm.at[idx_v.at[2 * c]], bufs[i].at[pl.ds(0, _HC)], gsems[i])
        d2 = pltpu.async_copy(
            x_hbm.at[idx_v.at[2 * c + 1]], bufs[i].at[pl.ds(_HC, _HC)],
            gsems[i])
        return (d1, d2)

    def out_slice(c):
        return out_hbm.at[b, pl.ds(lo + c * _CHUNK, _CHUNK)]

    gs = [None] * _NB
    stores = [None] * _NB

    for g in range(2):
        bisect_chunk(g)
        gs[g % _NB] = start_gather(g, g % _NB)

    pltpu.sync_copy(ml_hbm, mlv_v)
    max_len = mlv_v[...][0]
    cap = jnp.minimum(jnp.minimum(total, max_len), _L)
    nvalid = jnp.clip(cap - lo, 0, _HALF)

    def _mel():
        mel_v[...] = jnp.zeros((_LANES,), jnp.int32) + total
        pltpu.sync_copy(mel_v, mel_hbm.at[b])
    pl.when(half == 0)(_mel)

    for c in range(_NCHUNK):
        i = c % _NB
        g = c + 2
        if g < _NCHUNK:
            jbuf = g % _NB
            if stores[jbuf] is not None:
                stores[jbuf].wait()
            bisect_chunk(g)
            gs[jbuf] = start_gather(g, jbuf)
        gs[i][0].wait()
        gs[i][1].wait()
        nv_c = jnp.clip(nvalid - c * _CHUNK, 0, _CHUNK)

        def z_body(r, _, _i=i):
            zer = jnp.zeros((_LANES,), jnp.float32)
            for q in range(_D // _LANES):
                bufs[_i][r, pl.ds(q * _LANES, _LANES)] = zer
            return 0

        lax.fori_loop(nv_c, _CHUNK, z_body, 0)
        stores[i] = pltpu.async_copy(bufs[i], out_slice(c), ssems[i])
    for i in range(_NB):
        stores[i].wait()


def kernel(x, duration, max_len):
    x2 = x.reshape(_B * _T, _D)
    ml = jnp.full((_LANES,), max_len, dtype=jnp.int32)
    mesh = plsc.VectorSubcoreMesh(
        core_axis_name="c", subcore_axis_name="s",
        num_cores=_NC, num_subcores=_NS)
    f = pl.kernel(
        _lr_body,
        out_type=(
            jax.ShapeDtypeStruct((_B, _L, _D), jnp.float32),
            jax.ShapeDtypeStruct((_B, _LANES), jnp.int32),
        ),
        mesh=mesh,
        compiler_params=pltpu.CompilerParams(needs_layout_passes=False),
        scratch_types=[
            pltpu.VMEM((_T,), jnp.int32),
            pltpu.VMEM((_T + _LANES,), jnp.int32),
            pltpu.VMEM((_NCHUNK * 2, _CHUNK // 2), jnp.int32),
            pltpu.VMEM((_CHUNK, _D), jnp.float32),
            pltpu.VMEM((_CHUNK, _D), jnp.float32),
            pltpu.VMEM((_CHUNK, _D), jnp.float32),
            pltpu.VMEM((_CHUNK, _D), jnp.float32),
            pltpu.VMEM((_LANES,), jnp.int32),
            pltpu.VMEM((_LANES,), jnp.int32),
            pltpu.SemaphoreType.DMA,
            pltpu.SemaphoreType.DMA,
            pltpu.SemaphoreType.DMA,
            pltpu.SemaphoreType.DMA,
            pltpu.SemaphoreType.DMA,
            pltpu.SemaphoreType.DMA,
            pltpu.SemaphoreType.DMA,
            pltpu.SemaphoreType.DMA,
        ],
    )
    out, mel2d = f(x2, duration, ml)
    return out, mel2d[:, 0]
